# baseline (speedup 1.0000x reference)
import functools

import jax
import jax.numpy as jnp
from jax import lax
from jax.experimental import pallas as pl
from jax.experimental.pallas import tpu as pltpu
from jax.experimental.pallas import tpu_sc as plsc

TOKEN_DIM = 64
EMBED_DIM = 64

_NC = 2
_NS = 16
_NW = _NC * _NS

_IDX_ROW = 128
_K = 4


def _make_gather(n_rows: int, d: int):
    rows_per_w = n_rows // _NW
    chunks = rows_per_w // _K
    mesh = plsc.VectorSubcoreMesh(core_axis_name="c", subcore_axis_name="s")

    @functools.partial(
        pl.kernel,
        mesh=mesh,
        out_type=jax.ShapeDtypeStruct((n_rows, _IDX_ROW, d), jnp.float32),
        scratch_types=[
            pltpu.VMEM((_K, _IDX_ROW), jnp.int32),
            pltpu.VMEM((_K, _IDX_ROW, d), jnp.float32),
            pltpu.SemaphoreType.DMA,
        ],
        compiler_params=pltpu.CompilerParams(use_tc_tiling_on_sc=False),
    )
    def gather_k(idx_hbm, table_hbm, out_hbm, idx_v, rows_v, sem):
        wid = lax.axis_index("s") * _NC + lax.axis_index("c")
        base = wid * rows_per_w

        def body(g, carry):
            row = base + g * _K
            pltpu.sync_copy(idx_hbm.at[pl.ds(row, _K)], idx_v)
            cps = [
                pltpu.async_copy(table_hbm.at[idx_v.at[j]], rows_v.at[j], sem)
                for j in range(_K)
            ]
            for cp in cps:
                cp.wait()
            pltpu.sync_copy(rows_v, out_hbm.at[pl.ds(row, _K)])
            return carry

        lax.fori_loop(0, chunks, body, 0)

    return gather_k


def _mm_body(x_ref, w_ref, b_ref, o_ref):
    acc = jnp.dot(x_ref[...], w_ref[...], preferred_element_type=jnp.float32)
    o_ref[...] = jnp.maximum(acc + b_ref[...], 0.0)


def _matmul_relu(x, w, bias):
    n = x.shape[0]
    blk = 8192
    return pl.pallas_call(
        _mm_body,
        grid=(n // blk,),
        in_specs=[
            pl.BlockSpec((blk, TOKEN_DIM), lambda i: (i, 0)),
            pl.BlockSpec((TOKEN_DIM, EMBED_DIM), lambda i: (0, 0)),
            pl.BlockSpec((1, EMBED_DIM), lambda i: (0, 0)),
        ],
        out_specs=pl.BlockSpec((blk, EMBED_DIM), lambda i: (i, 0)),
        out_shape=jax.ShapeDtypeStruct((n, EMBED_DIM), jnp.float32),
    )(x, w, bias.reshape(1, EMBED_DIM))


def kernel(raw_seqs, embed_table, W, b):
    batch, seq_len = raw_seqs.shape
    n_total = batch * seq_len
    idx = raw_seqs.reshape(n_total // _IDX_ROW, _IDX_ROW).astype(jnp.int32)
    gathered = _make_gather(n_total // _IDX_ROW, TOKEN_DIM)(idx, embed_table)
    flat = gathered.reshape(n_total, TOKEN_DIM)
    out = _matmul_relu(flat, W, b)
    return out.reshape(batch, seq_len, EMBED_DIM)


# speedup vs baseline: 1.0391x; 1.0391x over previous
import functools

import jax
import jax.numpy as jnp
from jax import lax
from jax.experimental import pallas as pl
from jax.experimental.pallas import tpu as pltpu
from jax.experimental.pallas import tpu_sc as plsc

TOKEN_DIM = 64
EMBED_DIM = 64

_NC = 2
_NS = 16
_NW = _NC * _NS

_IDX_ROW = 128
_K = 4


_NBUF = 8


def _make_gather(n_rows: int, d: int):
    rows_per_w = n_rows // _NW
    assert rows_per_w % _NBUF == 0
    waves = rows_per_w // _NBUF - 1
    mesh = plsc.VectorSubcoreMesh(core_axis_name="c", subcore_axis_name="s")

    @functools.partial(
        pl.kernel,
        mesh=mesh,
        out_type=jax.ShapeDtypeStruct((n_rows, _IDX_ROW, d), jnp.float32),
        scratch_types=[
            pltpu.VMEM((rows_per_w, _IDX_ROW), jnp.int32),
            pltpu.VMEM((_NBUF, _IDX_ROW, d), jnp.float32),
            pltpu.SemaphoreType.DMA((_NBUF,)),
        ],
        compiler_params=pltpu.CompilerParams(use_tc_tiling_on_sc=False),
    )
    def gather_k(idx_hbm, table_hbm, out_hbm, idx_all, rows_v, sems):
        wid = lax.axis_index("s") * _NC + lax.axis_index("c")
        base = wid * rows_per_w
        pltpu.sync_copy(idx_hbm.at[pl.ds(base, rows_per_w)], idx_all)
        for s in range(_NBUF):
            pltpu.async_copy(table_hbm.at[idx_all.at[s]], rows_v.at[s], sems.at[s])

        def wave(w, carry):
            g0 = w * _NBUF
            for s in range(_NBUF):
                g = g0 + s
                pltpu.make_async_copy(
                    table_hbm.at[idx_all.at[s]], rows_v.at[s], sems.at[s]
                ).wait()
                pltpu.sync_copy(rows_v.at[s], out_hbm.at[base + g])
                pltpu.async_copy(
                    table_hbm.at[idx_all.at[g + _NBUF]], rows_v.at[s], sems.at[s]
                )
            return carry

        lax.fori_loop(0, waves, wave, 0)
        for s in range(_NBUF):
            g = waves * _NBUF + s
            pltpu.make_async_copy(
                table_hbm.at[idx_all.at[s]], rows_v.at[s], sems.at[s]
            ).wait()
            pltpu.sync_copy(rows_v.at[s], out_hbm.at[base + g])

    return gather_k


def _mm_body(x_ref, w_ref, b_ref, o_ref):
    acc = jnp.dot(x_ref[...], w_ref[...], preferred_element_type=jnp.float32)
    o_ref[...] = jnp.maximum(acc + b_ref[...], 0.0)


def _matmul_relu(x, w, bias):
    n = x.shape[0]
    blk = 8192
    return pl.pallas_call(
        _mm_body,
        grid=(n // blk,),
        in_specs=[
            pl.BlockSpec((blk, TOKEN_DIM), lambda i: (i, 0)),
            pl.BlockSpec((TOKEN_DIM, EMBED_DIM), lambda i: (0, 0)),
            pl.BlockSpec((1, EMBED_DIM), lambda i: (0, 0)),
        ],
        out_specs=pl.BlockSpec((blk, EMBED_DIM), lambda i: (i, 0)),
        out_shape=jax.ShapeDtypeStruct((n, EMBED_DIM), jnp.float32),
    )(x, w, bias.reshape(1, EMBED_DIM))


def kernel(raw_seqs, embed_table, W, b):
    batch, seq_len = raw_seqs.shape
    n_total = batch * seq_len
    idx = raw_seqs.reshape(n_total // _IDX_ROW, _IDX_ROW).astype(jnp.int32)
    gathered = _make_gather(n_total // _IDX_ROW, TOKEN_DIM)(idx, embed_table)
    flat = gathered.reshape(n_total, TOKEN_DIM)
    out = _matmul_relu(flat, W, b)
    return out.reshape(batch, seq_len, EMBED_DIM)


# speedup vs baseline: 1.1721x; 1.1280x over previous
import functools

import jax
import jax.numpy as jnp
from jax import lax
from jax.experimental import pallas as pl
from jax.experimental.pallas import tpu as pltpu
from jax.experimental.pallas import tpu_sc as plsc

TOKEN_DIM = 64
EMBED_DIM = 64

_NC = 2
_NS = 16
_NW = _NC * _NS

_IDX_ROW = 128
_NBUF = 8

_BLKA = 8192


def _transform_body(xt_ref, w_ref, b_ref, o_ref):
    xt = xt_ref[...]
    w = w_ref[...]
    y = jax.lax.dot_general(
        xt, w, (((0,), (0,)), ((), ())),
        preferred_element_type=jnp.float32,
    )
    y = jnp.maximum(y + b_ref[...], 0.0)
    yu = jax.lax.bitcast_convert_type(y.astype(jnp.bfloat16), jnp.uint16)
    ye = yu[:, : EMBED_DIM // 2].astype(jnp.uint32)
    yo = yu[:, EMBED_DIM // 2 :].astype(jnp.uint32)
    o_ref[...] = jax.lax.bitcast_convert_type(
        ye | (yo << 16), jnp.float32
    )


def _transform_table(table_t, w, bias):
    v = table_t.shape[1]
    nblk = (v + _BLKA - 1) // _BLKA
    return pl.pallas_call(
        _transform_body,
        grid=(nblk,),
        in_specs=[
            pl.BlockSpec((TOKEN_DIM, _BLKA), lambda i: (0, i)),
            pl.BlockSpec((TOKEN_DIM, EMBED_DIM), lambda i: (0, 0)),
            pl.BlockSpec((1, EMBED_DIM), lambda i: (0, 0)),
        ],
        out_specs=pl.BlockSpec((_BLKA, EMBED_DIM // 2), lambda i: (i, 0)),
        out_shape=jax.ShapeDtypeStruct((nblk * _BLKA, EMBED_DIM // 2), jnp.float32),
    )(table_t, w, bias.reshape(1, EMBED_DIM))


def _make_gather(n_rows: int, d: int):
    rows_per_w = n_rows // _NW
    assert rows_per_w % _NBUF == 0
    waves = rows_per_w // _NBUF - 1
    mesh = plsc.VectorSubcoreMesh(core_axis_name="c", subcore_axis_name="s")

    @functools.partial(
        pl.kernel,
        mesh=mesh,
        out_type=jax.ShapeDtypeStruct((n_rows, _IDX_ROW, d), jnp.float32),
        scratch_types=[
            pltpu.VMEM((rows_per_w, _IDX_ROW), jnp.int32),
            pltpu.VMEM((_NBUF, _IDX_ROW, d), jnp.float32),
            pltpu.SemaphoreType.DMA((_NBUF,)),
        ],
        compiler_params=pltpu.CompilerParams(use_tc_tiling_on_sc=False),
    )
    def gather_k(idx_hbm, table_hbm, out_hbm, idx_all, rows_v, sems):
        wid = lax.axis_index("s") * _NC + lax.axis_index("c")
        base = wid * rows_per_w
        pltpu.sync_copy(idx_hbm.at[pl.ds(base, rows_per_w)], idx_all)
        for s in range(_NBUF):
            pltpu.async_copy(table_hbm.at[idx_all.at[s]], rows_v.at[s], sems.at[s])

        def wave(wv, carry):
            g0 = wv * _NBUF
            for s in range(_NBUF):
                g = g0 + s
                pltpu.make_async_copy(
                    table_hbm.at[idx_all.at[s]], rows_v.at[s], sems.at[s]
                ).wait()
                pltpu.sync_copy(rows_v.at[s], out_hbm.at[base + g])
                pltpu.async_copy(
                    table_hbm.at[idx_all.at[g + _NBUF]], rows_v.at[s], sems.at[s]
                )
            return carry

        lax.fori_loop(0, waves, wave, 0)
        for s in range(_NBUF):
            g = waves * _NBUF + s
            pltpu.make_async_copy(
                table_hbm.at[idx_all.at[s]], rows_v.at[s], sems.at[s]
            ).wait()
            pltpu.sync_copy(rows_v.at[s], out_hbm.at[base + g])

    return gather_k


def _format_body(x_ref, o_ref):
    x = jax.lax.bitcast_convert_type(x_ref[0], jnp.uint32)
    lo = jax.lax.bitcast_convert_type(
        (x & 0xFFFF).astype(jnp.uint16), jnp.bfloat16
    ).astype(jnp.float32)
    hi = jax.lax.bitcast_convert_type(
        (x >> 16).astype(jnp.uint16), jnp.bfloat16
    ).astype(jnp.float32)
    o_ref[0] = jnp.stack([lo.T, hi.T], axis=1).reshape(TOKEN_DIM, x.shape[0])


def _format_out(inter, seq_len, batch):
    return pl.pallas_call(
        _format_body,
        grid=(seq_len,),
        in_specs=[pl.BlockSpec((1, batch, EMBED_DIM // 2), lambda s: (s, 0, 0))],
        out_specs=pl.BlockSpec((1, EMBED_DIM, batch), lambda s: (s, 0, 0)),
        out_shape=jax.ShapeDtypeStruct((seq_len, EMBED_DIM, batch), jnp.float32),
    )(inter)


def kernel(raw_seqs, embed_table, W, b):
    batch, seq_len = raw_seqs.shape
    n_total = batch * seq_len
    perm = jnp.concatenate(
        [jnp.arange(0, EMBED_DIM, 2), jnp.arange(1, EMBED_DIM, 2)]
    )
    w2 = W[:, perm]
    b2 = b[perm]
    table2 = _transform_table(embed_table.T, w2, b2)
    idx_t = jnp.transpose(raw_seqs).reshape(
        n_total // _IDX_ROW, _IDX_ROW
    ).astype(jnp.int32)
    gathered = _make_gather(n_total // _IDX_ROW, EMBED_DIM // 2)(idx_t, table2)
    inter = gathered.reshape(seq_len, batch, EMBED_DIM // 2)
    out3 = _format_out(inter, seq_len, batch)
    return jnp.transpose(out3, (2, 0, 1))


# speedup vs baseline: 1.7473x; 1.4908x over previous
import functools

import jax
import jax.numpy as jnp
from jax import lax
from jax.experimental import pallas as pl
from jax.experimental.pallas import tpu as pltpu
from jax.experimental.pallas import tpu_sc as plsc

TOKEN_DIM = 64
EMBED_DIM = 64

_NC = 2
_NS = 16
_NW = _NC * _NS

_IDX_ROW = 128
_NBUF = 8

_BLKA = 8192


def _transform_body(xt_ref, w_ref, b_ref, o_ref):
    xt = xt_ref[...]
    w = w_ref[...]
    y = jax.lax.dot_general(
        xt, w, (((0,), (0,)), ((), ())),
        preferred_element_type=jnp.float32,
    )
    y = jnp.maximum(y + b_ref[...], 0.0)
    yu = jax.lax.bitcast_convert_type(y.astype(jnp.bfloat16), jnp.uint16)
    ye = yu[:, : EMBED_DIM // 2].astype(jnp.uint32)
    yo = yu[:, EMBED_DIM // 2 :].astype(jnp.uint32)
    packed = jax.lax.bitcast_convert_type(
        ye | (yo << 16), jnp.float32
    )
    q = y.shape[0] // 4
    o_ref[...] = jnp.concatenate(
        [packed[0:q], packed[q : 2 * q], packed[2 * q : 3 * q], packed[3 * q :]],
        axis=1,
    )


def _transform_table(table_t, w, bias):
    v = table_t.shape[1]
    nblk = (v + _BLKA - 1) // _BLKA
    return pl.pallas_call(
        _transform_body,
        grid=(nblk,),
        in_specs=[
            pl.BlockSpec((TOKEN_DIM, _BLKA), lambda i: (0, i)),
            pl.BlockSpec((TOKEN_DIM, EMBED_DIM), lambda i: (0, 0)),
            pl.BlockSpec((1, EMBED_DIM), lambda i: (0, 0)),
        ],
        out_specs=pl.BlockSpec((_BLKA // 4, 128), lambda i: (i, 0)),
        out_shape=jax.ShapeDtypeStruct((nblk * _BLKA // 4, 128), jnp.float32),
    )(table_t, w, bias.reshape(1, EMBED_DIM))


def _make_gather(n_rows: int, d: int):
    rows_per_w = n_rows // _NW
    assert rows_per_w % _NBUF == 0
    waves = rows_per_w // _NBUF - 1
    mesh = plsc.VectorSubcoreMesh(core_axis_name="c", subcore_axis_name="s")

    @functools.partial(
        pl.kernel,
        mesh=mesh,
        out_type=jax.ShapeDtypeStruct((n_rows, _IDX_ROW, d), jnp.float32),
        scratch_types=[
            pltpu.VMEM((rows_per_w, _IDX_ROW), jnp.int32),
            pltpu.VMEM((_NBUF, _IDX_ROW, d), jnp.float32),
            pltpu.SemaphoreType.DMA((_NBUF,)),
        ],
        compiler_params=pltpu.CompilerParams(use_tc_tiling_on_sc=False),
    )
    def gather_k(idx_hbm, table_hbm, out_hbm, idx_all, rows_v, sems):
        wid = lax.axis_index("s") * _NC + lax.axis_index("c")
        base = wid * rows_per_w
        pltpu.sync_copy(idx_hbm.at[pl.ds(base, rows_per_w)], idx_all)
        for s in range(_NBUF):
            pltpu.async_copy(table_hbm.at[idx_all.at[s]], rows_v.at[s], sems.at[s])

        def wave(wv, carry):
            g0 = wv * _NBUF
            for s in range(_NBUF):
                g = g0 + s
                pltpu.make_async_copy(
                    table_hbm.at[idx_all.at[s]], rows_v.at[s], sems.at[s]
                ).wait()
                pltpu.sync_copy(rows_v.at[s], out_hbm.at[base + g])
                pltpu.async_copy(
                    table_hbm.at[idx_all.at[g + _NBUF]], rows_v.at[s], sems.at[s]
                )
            return carry

        lax.fori_loop(0, waves, wave, 0)
        for s in range(_NBUF):
            g = waves * _NBUF + s
            pltpu.make_async_copy(
                table_hbm.at[idx_all.at[s]], rows_v.at[s], sems.at[s]
            ).wait()
            pltpu.sync_copy(rows_v.at[s], out_hbm.at[base + g])

    return gather_k


def _format_body(x_ref, o_ref):
    xw = jax.lax.bitcast_convert_type(x_ref[...], jnp.uint32)
    rows = xw.shape[0]
    w = EMBED_DIM // 2
    los, his = [], []
    for q in range(4):
        xq = xw[:, q * w : (q + 1) * w]
        los.append(
            jax.lax.bitcast_convert_type(
                (xq & 0xFFFF).astype(jnp.uint16), jnp.bfloat16
            ).astype(jnp.float32).T
        )
        his.append(
            jax.lax.bitcast_convert_type(
                (xq >> 16).astype(jnp.uint16), jnp.bfloat16
            ).astype(jnp.float32).T
        )
    lo = jnp.concatenate(los, axis=1)
    hi = jnp.concatenate(his, axis=1)
    o_ref[0] = jnp.stack([lo, hi], axis=1).reshape(TOKEN_DIM, rows * 4)


def _format_out(inter, seq_len, batch):
    return pl.pallas_call(
        _format_body,
        grid=(seq_len,),
        in_specs=[pl.BlockSpec((batch // 4, 128), lambda s: (s, 0))],
        out_specs=pl.BlockSpec((1, EMBED_DIM, batch), lambda s: (s, 0, 0)),
        out_shape=jax.ShapeDtypeStruct((seq_len, EMBED_DIM, batch), jnp.float32),
    )(inter)


def kernel(raw_seqs, embed_table, W, b):
    batch, seq_len = raw_seqs.shape
    n_total = batch * seq_len
    perm = jnp.concatenate(
        [jnp.arange(0, EMBED_DIM, 2), jnp.arange(1, EMBED_DIM, 2)]
    )
    w2 = W[:, perm]
    b2 = b[perm]
    table2 = _transform_table(embed_table.T, w2, b2)
    table4 = table2.reshape(table2.shape[0] * 4, EMBED_DIM // 2)
    v = raw_seqs.astype(jnp.int32)
    i_blk = v // _BLKA
    j_loc = v % _BLKA
    qtr = _BLKA // 4
    vm = (i_blk * qtr + j_loc % qtr) * 4 + j_loc // qtr
    idx_t = (
        jnp.transpose(vm)
        .reshape(seq_len, 4, batch // 4)
        .transpose(0, 2, 1)
        .reshape(n_total // _IDX_ROW, _IDX_ROW)
    )
    gathered = _make_gather(n_total // _IDX_ROW, EMBED_DIM // 2)(idx_t, table4)
    inter = gathered.reshape(n_total // 4, 128)
    out3 = _format_out(inter, seq_len, batch)
    return jnp.transpose(out3, (2, 0, 1))


# speedup vs baseline: 2.0407x; 1.1680x over previous
import functools

import jax
import jax.numpy as jnp
from jax import lax
from jax.experimental import pallas as pl
from jax.experimental.pallas import tpu as pltpu
from jax.experimental.pallas import tpu_sc as plsc

TOKEN_DIM = 64
EMBED_DIM = 64

_NC = 2
_NS = 16
_NW = _NC * _NS

_IDX_ROW = 128
_NBUF = 8
_LANES = 16

_BLKA = 16384
_BLKA_BITS = 14


def _transform_body(xt_ref, w_ref, b_ref, o_ref):
    xt = xt_ref[...]
    w = w_ref[...]
    y = jax.lax.dot_general(
        xt, w, (((0,), (0,)), ((), ())),
        preferred_element_type=jnp.float32,
    )
    y = jnp.maximum(y + b_ref[...], 0.0)
    yu = jax.lax.bitcast_convert_type(y.astype(jnp.bfloat16), jnp.uint16)
    ye = yu[:, : EMBED_DIM // 2].astype(jnp.uint32)
    yo = yu[:, EMBED_DIM // 2 :].astype(jnp.uint32)
    packed = jax.lax.bitcast_convert_type(
        ye | (yo << 16), jnp.float32
    )
    q = y.shape[0] // 4
    o_ref[...] = jnp.concatenate(
        [packed[0:q], packed[q : 2 * q], packed[2 * q : 3 * q], packed[3 * q :]],
        axis=1,
    )


def _transform_table(table_t, w, bias):
    v = table_t.shape[1]
    nblk = (v + _BLKA - 1) // _BLKA
    return pl.pallas_call(
        _transform_body,
        grid=(nblk,),
        in_specs=[
            pl.BlockSpec((TOKEN_DIM, _BLKA), lambda i: (0, i)),
            pl.BlockSpec((TOKEN_DIM, EMBED_DIM), lambda i: (0, 0)),
            pl.BlockSpec((1, EMBED_DIM), lambda i: (0, 0)),
        ],
        out_specs=pl.BlockSpec((_BLKA // 4, 128), lambda i: (i, 0)),
        out_shape=jax.ShapeDtypeStruct((nblk * _BLKA // 4, 128), jnp.float32),
    )(table_t, w, bias.reshape(1, EMBED_DIM))


def _make_gather(seq_len: int, batch: int, d: int):
    n_rows = seq_len * batch // _IDX_ROW
    assert seq_len % _NBUF == 0
    waves = seq_len // _NBUF - 1
    seg = _IDX_ROW // 4
    mesh = plsc.VectorSubcoreMesh(core_axis_name="c", subcore_axis_name="s")

    @functools.partial(
        pl.kernel,
        mesh=mesh,
        out_type=jax.ShapeDtypeStruct((n_rows, _IDX_ROW, d), jnp.float32),
        scratch_types=[
            pltpu.VMEM((seq_len, _IDX_ROW), jnp.int32),
            pltpu.VMEM((1, _IDX_ROW), jnp.int32),
            pltpu.VMEM((_NBUF, _IDX_ROW), jnp.int32),
            pltpu.VMEM((_NBUF, _IDX_ROW, d), jnp.float32),
            pltpu.SemaphoreType.DMA((_NBUF,)),
        ],
        compiler_params=pltpu.CompilerParams(
            use_tc_tiling_on_sc=False, needs_layout_passes=False
        ),
    )
    def gather_k(idxt_hbm, table_hbm, out_hbm, idx_all, perm, idx2, rows_v, sems):
        wid = lax.axis_index("s") * _NC + lax.axis_index("c")

        for j in range(4):
            pltpu.sync_copy(
                idxt_hbm.at[:, pl.ds(j * (batch // 4) + wid * seg, seg)],
                idx_all.at[:, pl.ds(j * seg, seg)],
            )
        for i in range(_IDX_ROW // _LANES):
            lam = lax.iota(jnp.int32, _LANES) + i * _LANES
            perm[0, pl.ds(i * _LANES, _LANES)] = seg * (lam & 3) + (lam >> 2)

        def prep(slot, s):
            s_vec = jnp.full((_LANES,), s, jnp.int32)
            for i in range(_IDX_ROW // _LANES):
                pv = perm[0, pl.ds(i * _LANES, _LANES)]
                v = plsc.load_gather(idx_all, [s_vec, pv])
                j = v & (_BLKA - 1)
                ib = v >> _BLKA_BITS
                vm = (
                    (ib << _BLKA_BITS)
                    + ((j & (_BLKA // 4 - 1)) << 2)
                    + (j >> (_BLKA_BITS - 2))
                )
                idx2[slot, pl.ds(i * _LANES, _LANES)] = vm

        for s in range(_NBUF):
            prep(s, s)
            pltpu.async_copy(table_hbm.at[idx2.at[s]], rows_v.at[s], sems.at[s])

        def wave(wv, carry):
            g0 = wv * _NBUF
            for slot in range(_NBUF):
                g = g0 + slot
                pltpu.make_async_copy(
                    table_hbm.at[idx2.at[slot]], rows_v.at[slot], sems.at[slot]
                ).wait()
                pltpu.sync_copy(rows_v.at[slot], out_hbm.at[g * _NW + wid])
                prep(slot, g + _NBUF)
                pltpu.async_copy(
                    table_hbm.at[idx2.at[slot]], rows_v.at[slot], sems.at[slot]
                )
            return carry

        lax.fori_loop(0, waves, wave, 0)
        for slot in range(_NBUF):
            g = waves * _NBUF + slot
            pltpu.make_async_copy(
                table_hbm.at[idx2.at[slot]], rows_v.at[slot], sems.at[slot]
            ).wait()
            pltpu.sync_copy(rows_v.at[slot], out_hbm.at[g * _NW + wid])

    return gather_k


def _format_body(x_ref, o_ref):
    xw = jax.lax.bitcast_convert_type(x_ref[...], jnp.uint32)
    w = EMBED_DIM // 2
    x = jnp.concatenate(
        [xw[:, q * w : (q + 1) * w].T for q in range(4)], axis=1
    )
    lo = jax.lax.bitcast_convert_type(
        (x & 0xFFFF).astype(jnp.uint16), jnp.bfloat16
    ).astype(jnp.float32)
    hi = jax.lax.bitcast_convert_type(
        (x >> 16).astype(jnp.uint16), jnp.bfloat16
    ).astype(jnp.float32)
    o_ref[0] = jnp.stack([lo, hi], axis=1).reshape(TOKEN_DIM, x.shape[1])


def _format_out(inter, seq_len, batch):
    return pl.pallas_call(
        _format_body,
        grid=(seq_len,),
        in_specs=[pl.BlockSpec((batch // 4, 128), lambda s: (s, 0))],
        out_specs=pl.BlockSpec((1, EMBED_DIM, batch), lambda s: (s, 0, 0)),
        out_shape=jax.ShapeDtypeStruct((seq_len, EMBED_DIM, batch), jnp.float32),
    )(inter)


def kernel(raw_seqs, embed_table, W, b):
    batch, seq_len = raw_seqs.shape
    n_total = batch * seq_len
    perm = jnp.concatenate(
        [jnp.arange(0, EMBED_DIM, 2), jnp.arange(1, EMBED_DIM, 2)]
    )
    w2 = W[:, perm]
    b2 = b[perm]
    table2 = _transform_table(embed_table.T, w2, b2)
    table4 = table2.reshape(table2.shape[0] * 4, EMBED_DIM // 2)
    idx_t = jnp.transpose(raw_seqs).astype(jnp.int32)
    gathered = _make_gather(seq_len, batch, EMBED_DIM // 2)(idx_t, table4)
    inter = gathered.reshape(n_total // 4, 128)
    out3 = _format_out(inter, seq_len, batch)
    return jnp.transpose(out3, (2, 0, 1))


# speedup vs baseline: 2.6672x; 1.3069x over previous
import functools

import jax
import jax.numpy as jnp
from jax import lax
from jax.experimental import pallas as pl
from jax.experimental.pallas import tpu as pltpu
from jax.experimental.pallas import tpu_sc as plsc

TOKEN_DIM = 64
EMBED_DIM = 64

_NC = 2
_NS = 16
_NW = _NC * _NS

_IDX_ROW = 128
_NBUF = 8
_LANES = 16

_BLKA = 16384
_BLKA_BITS = 14


def _transform_body(xt_ref, w_ref, b_ref, o_ref):
    xt = xt_ref[...]
    w = w_ref[...]
    y = jax.lax.dot_general(
        xt, w, (((0,), (0,)), ((), ())),
        preferred_element_type=jnp.float32,
    )
    y = jnp.maximum(y + b_ref[...], 0.0)
    yu = jax.lax.bitcast_convert_type(y.astype(jnp.bfloat16), jnp.uint16)
    ye = yu[:, : EMBED_DIM // 2].astype(jnp.uint32)
    yo = yu[:, EMBED_DIM // 2 :].astype(jnp.uint32)
    packed = jax.lax.bitcast_convert_type(
        ye | (yo << 16), jnp.float32
    )
    q = y.shape[0] // 4
    o_ref[...] = jnp.concatenate(
        [packed[0:q], packed[q : 2 * q], packed[2 * q : 3 * q], packed[3 * q :]],
        axis=1,
    )


def _transform_table(table_t, w, bias):
    v = table_t.shape[1]
    nblk = (v + _BLKA - 1) // _BLKA
    return pl.pallas_call(
        _transform_body,
        grid=(nblk,),
        in_specs=[
            pl.BlockSpec((TOKEN_DIM, _BLKA), lambda i: (0, i)),
            pl.BlockSpec((TOKEN_DIM, EMBED_DIM), lambda i: (0, 0)),
            pl.BlockSpec((1, EMBED_DIM), lambda i: (0, 0)),
        ],
        out_specs=pl.BlockSpec((_BLKA // 4, 128), lambda i: (i, 0)),
        out_shape=jax.ShapeDtypeStruct((nblk * _BLKA // 4, 128), jnp.float32),
        compiler_params=pltpu.CompilerParams(fuse_transposed_lhs_in_matmul=True),
    )(table_t, w, bias.reshape(1, EMBED_DIM))


def _make_gather(seq_len: int, batch: int, d: int):
    n_rows = seq_len * batch // _IDX_ROW
    assert seq_len % _NBUF == 0
    waves = seq_len // _NBUF - 1
    seg = _IDX_ROW // 4
    mesh = plsc.VectorSubcoreMesh(core_axis_name="c", subcore_axis_name="s")

    @functools.partial(
        pl.kernel,
        mesh=mesh,
        out_type=jax.ShapeDtypeStruct((n_rows, _IDX_ROW, d), jnp.float32),
        scratch_types=[
            pltpu.VMEM((seq_len, _IDX_ROW), jnp.int32),
            pltpu.VMEM((1, _IDX_ROW), jnp.int32),
            pltpu.VMEM((_NBUF, _IDX_ROW), jnp.int32),
            pltpu.VMEM((_NBUF, _IDX_ROW, d), jnp.float32),
            pltpu.SemaphoreType.DMA((_NBUF,)),
        ],
        compiler_params=pltpu.CompilerParams(
            use_tc_tiling_on_sc=False, needs_layout_passes=False
        ),
    )
    def gather_k(idxt_hbm, table_hbm, out_hbm, idx_all, perm, idx2, rows_v, sems):
        wid = lax.axis_index("s") * _NC + lax.axis_index("c")

        for j in range(4):
            pltpu.sync_copy(
                idxt_hbm.at[:, pl.ds(j * (batch // 4) + wid * seg, seg)],
                idx_all.at[:, pl.ds(j * seg, seg)],
            )
        for i in range(_IDX_ROW // _LANES):
            lam = lax.iota(jnp.int32, _LANES) + i * _LANES
            perm[0, pl.ds(i * _LANES, _LANES)] = seg * (lam & 3) + (lam >> 2)

        def prep(slot, s):
            s_vec = jnp.full((_LANES,), s, jnp.int32)
            for i in range(_IDX_ROW // _LANES):
                pv = perm[0, pl.ds(i * _LANES, _LANES)]
                v = plsc.load_gather(idx_all, [s_vec, pv])
                j = v & (_BLKA - 1)
                ib = v >> _BLKA_BITS
                vm = (
                    (ib << _BLKA_BITS)
                    + ((j & (_BLKA // 4 - 1)) << 2)
                    + (j >> (_BLKA_BITS - 2))
                )
                idx2[slot, pl.ds(i * _LANES, _LANES)] = vm

        for s in range(_NBUF):
            prep(s, s)
            pltpu.async_copy(table_hbm.at[idx2.at[s]], rows_v.at[s], sems.at[s])

        def wave(wv, carry):
            g0 = wv * _NBUF
            for slot in range(_NBUF):
                g = g0 + slot
                pltpu.make_async_copy(
                    table_hbm.at[idx2.at[slot]], rows_v.at[slot], sems.at[slot]
                ).wait()
                pltpu.sync_copy(rows_v.at[slot], out_hbm.at[g * _NW + wid])
                prep(slot, g + _NBUF)
                pltpu.async_copy(
                    table_hbm.at[idx2.at[slot]], rows_v.at[slot], sems.at[slot]
                )
            return carry

        lax.fori_loop(0, waves, wave, 0)
        for slot in range(_NBUF):
            g = waves * _NBUF + slot
            pltpu.make_async_copy(
                table_hbm.at[idx2.at[slot]], rows_v.at[slot], sems.at[slot]
            ).wait()
            pltpu.sync_copy(rows_v.at[slot], out_hbm.at[g * _NW + wid])

    return gather_k


def _format_body(x_ref, o_ref):
    xw = jax.lax.bitcast_convert_type(x_ref[...], jnp.uint32)
    xt = xw.T
    w = EMBED_DIM // 2
    x = jnp.concatenate(
        [xt[q * w : (q + 1) * w] for q in range(4)], axis=1
    )
    lo = jax.lax.bitcast_convert_type(x << 16, jnp.float32)
    hi = jax.lax.bitcast_convert_type(
        x & jnp.uint32(0xFFFF0000), jnp.float32
    )
    o_ref[0] = jnp.concatenate([lo, hi], axis=0)


def _format_out(inter, seq_len, batch):
    return pl.pallas_call(
        _format_body,
        grid=(seq_len,),
        in_specs=[pl.BlockSpec((batch // 4, 128), lambda s: (s, 0))],
        out_specs=pl.BlockSpec((1, EMBED_DIM, batch), lambda s: (s, 0, 0)),
        out_shape=jax.ShapeDtypeStruct((seq_len, EMBED_DIM, batch), jnp.float32),
    )(inter)


def kernel(raw_seqs, embed_table, W, b):
    batch, seq_len = raw_seqs.shape
    n_total = batch * seq_len
    table2 = _transform_table(embed_table.T, W, b)
    table4 = table2.reshape(table2.shape[0] * 4, EMBED_DIM // 2)
    idx_t = jnp.transpose(raw_seqs).astype(jnp.int32)
    gathered = _make_gather(seq_len, batch, EMBED_DIM // 2)(idx_t, table4)
    inter = gathered.reshape(n_total // 4, 128)
    out3 = _format_out(inter, seq_len, batch)
    return jnp.transpose(out3, (2, 0, 1))


# speedup vs baseline: 2.6727x; 1.0021x over previous
import functools

import jax
import jax.numpy as jnp
from jax import lax
from jax.experimental import pallas as pl
from jax.experimental.pallas import tpu as pltpu
from jax.experimental.pallas import tpu_sc as plsc

TOKEN_DIM = 64
EMBED_DIM = 64

_NC = 2
_NS = 16
_NW = _NC * _NS

_IDX_ROW = 128
_NBUF = 8
_LANES = 16

_BLKA = 32768
_BLKA_BITS = 15


def _transform_body(xt_ref, w_ref, b_ref, o_ref):
    xt = xt_ref[...]
    w = w_ref[...]
    y = jax.lax.dot_general(
        xt, w, (((0,), (0,)), ((), ())),
        preferred_element_type=jnp.float32,
    )
    y = jnp.maximum(y + b_ref[...], 0.0)
    yu = jax.lax.bitcast_convert_type(y.astype(jnp.bfloat16), jnp.uint16)
    ye = yu[:, : EMBED_DIM // 2].astype(jnp.uint32)
    yo = yu[:, EMBED_DIM // 2 :].astype(jnp.uint32)
    packed = jax.lax.bitcast_convert_type(
        ye | (yo << 16), jnp.float32
    )
    q = y.shape[0] // 4
    o_ref[...] = jnp.concatenate(
        [packed[0:q], packed[q : 2 * q], packed[2 * q : 3 * q], packed[3 * q :]],
        axis=1,
    )


def _transform_table(table_t, w, bias):
    v = table_t.shape[1]
    nblk = (v + _BLKA - 1) // _BLKA
    return pl.pallas_call(
        _transform_body,
        grid=(nblk,),
        in_specs=[
            pl.BlockSpec((TOKEN_DIM, _BLKA), lambda i: (0, i)),
            pl.BlockSpec((TOKEN_DIM, EMBED_DIM), lambda i: (0, 0)),
            pl.BlockSpec((1, EMBED_DIM), lambda i: (0, 0)),
        ],
        out_specs=pl.BlockSpec((_BLKA // 4, 128), lambda i: (i, 0)),
        out_shape=jax.ShapeDtypeStruct((nblk * _BLKA // 4, 128), jnp.float32),
        compiler_params=pltpu.CompilerParams(fuse_transposed_lhs_in_matmul=True),
    )(table_t, w, bias.reshape(1, EMBED_DIM))


def _make_gather(seq_len: int, batch: int, d: int):
    n_rows = seq_len * batch // _IDX_ROW
    assert seq_len % _NBUF == 0
    waves = seq_len // _NBUF - 1
    seg = _IDX_ROW // 4
    mesh = plsc.VectorSubcoreMesh(core_axis_name="c", subcore_axis_name="s")

    @functools.partial(
        pl.kernel,
        mesh=mesh,
        out_type=jax.ShapeDtypeStruct((n_rows, _IDX_ROW, d), jnp.float32),
        scratch_types=[
            pltpu.VMEM((seq_len, _IDX_ROW), jnp.int32),
            pltpu.VMEM((1, _IDX_ROW), jnp.int32),
            pltpu.VMEM((_NBUF, _IDX_ROW), jnp.int32),
            pltpu.VMEM((_NBUF, _IDX_ROW, d), jnp.float32),
            pltpu.SemaphoreType.DMA((_NBUF,)),
        ],
        compiler_params=pltpu.CompilerParams(
            use_tc_tiling_on_sc=False, needs_layout_passes=False
        ),
    )
    def gather_k(idxt_hbm, table_hbm, out_hbm, idx_all, perm, idx2, rows_v, sems):
        wid = lax.axis_index("s") * _NC + lax.axis_index("c")

        for j in range(4):
            pltpu.sync_copy(
                idxt_hbm.at[:, pl.ds(j * (batch // 4) + wid * seg, seg)],
                idx_all.at[:, pl.ds(j * seg, seg)],
            )
        for i in range(_IDX_ROW // _LANES):
            lam = lax.iota(jnp.int32, _LANES) + i * _LANES
            perm[0, pl.ds(i * _LANES, _LANES)] = seg * (lam & 3) + (lam >> 2)

        def prep(slot, s):
            s_vec = jnp.full((_LANES,), s, jnp.int32)
            for i in range(_IDX_ROW // _LANES):
                pv = perm[0, pl.ds(i * _LANES, _LANES)]
                v = plsc.load_gather(idx_all, [s_vec, pv])
                j = v & (_BLKA - 1)
                ib = v >> _BLKA_BITS
                vm = (
                    (ib << _BLKA_BITS)
                    + ((j & (_BLKA // 4 - 1)) << 2)
                    + (j >> (_BLKA_BITS - 2))
                )
                idx2[slot, pl.ds(i * _LANES, _LANES)] = vm

        for s in range(_NBUF):
            prep(s, s)
            pltpu.async_copy(table_hbm.at[idx2.at[s]], rows_v.at[s], sems.at[s])

        def wave(wv, carry):
            g0 = wv * _NBUF
            for slot in range(_NBUF):
                g = g0 + slot
                pltpu.make_async_copy(
                    table_hbm.at[idx2.at[slot]], rows_v.at[slot], sems.at[slot]
                ).wait()
                pltpu.sync_copy(rows_v.at[slot], out_hbm.at[g * _NW + wid])
                prep(slot, g + _NBUF)
                pltpu.async_copy(
                    table_hbm.at[idx2.at[slot]], rows_v.at[slot], sems.at[slot]
                )
            return carry

        lax.fori_loop(0, waves, wave, 0)
        for slot in range(_NBUF):
            g = waves * _NBUF + slot
            pltpu.make_async_copy(
                table_hbm.at[idx2.at[slot]], rows_v.at[slot], sems.at[slot]
            ).wait()
            pltpu.sync_copy(rows_v.at[slot], out_hbm.at[g * _NW + wid])

    return gather_k


def _format_body(x_ref, o_ref):
    xw = jax.lax.bitcast_convert_type(x_ref[...], jnp.uint32)
    xt = xw.T
    w = EMBED_DIM // 2
    x = jnp.concatenate(
        [xt[q * w : (q + 1) * w] for q in range(4)], axis=1
    )
    lo = jax.lax.bitcast_convert_type(x << 16, jnp.float32)
    hi = jax.lax.bitcast_convert_type(
        x & jnp.uint32(0xFFFF0000), jnp.float32
    )
    o_ref[0] = jnp.concatenate([lo, hi], axis=0)


def _format_out(inter, seq_len, batch):
    return pl.pallas_call(
        _format_body,
        grid=(seq_len,),
        in_specs=[pl.BlockSpec((batch // 4, 128), lambda s: (s, 0))],
        out_specs=pl.BlockSpec((1, EMBED_DIM, batch), lambda s: (s, 0, 0)),
        out_shape=jax.ShapeDtypeStruct((seq_len, EMBED_DIM, batch), jnp.float32),
    )(inter)


def kernel(raw_seqs, embed_table, W, b):
    batch, seq_len = raw_seqs.shape
    n_total = batch * seq_len
    table2 = _transform_table(embed_table.T, W, b)
    table4 = table2.reshape(table2.shape[0] * 4, EMBED_DIM // 2)
    idx_t = jnp.transpose(raw_seqs).astype(jnp.int32)
    gathered = _make_gather(seq_len, batch, EMBED_DIM // 2)(idx_t, table4)
    inter = gathered.reshape(n_total // 4, 128)
    out3 = _format_out(inter, seq_len, batch)
    return jnp.transpose(out3, (2, 0, 1))
